# Initial kernel scaffold; baseline (speedup 1.0000x reference)
import functools

import jax
import jax.numpy as jnp
from jax import lax
from jax.experimental import pallas as pl
from jax.experimental.pallas import tpu as pltpu
from jax.experimental.pallas import tpu_sc as plsc

_N = 10000
_E = 160000
_IN = 256
_H = 512
_OUT = 256

_R = 1000
_G = _N // _R

_NS = 16
_NC = 2
_CHUNK = 128
_PROP_CH = 80
_EPAD = _NS * _PROP_CH * _CHUNK
_DEG_CH = _EPAD // (_NC * _NS * _CHUNK)
_RW = _N // _NS
_NPAD = 10016


_MESH = plsc.VectorSubcoreMesh(core_axis_name="c", subcore_axis_name="s")


@functools.partial(
    pl.kernel,
    out_type=jax.ShapeDtypeStruct((_NC * _N, 16), jnp.float32),
    mesh=_MESH,
    scratch_types=[
        pltpu.VMEM((_CHUNK,), jnp.int32),
        pltpu.VMEM((_CHUNK, 16), jnp.float32),
        pltpu.VMEM_SHARED((_NPAD, 16), jnp.float32),
    ],
)
def _deg_kernel(dst_hbm, zeros_hbm, ones_hbm, out_hbm, idx_v, ones_v, deg_sh):
    c = lax.axis_index("c")
    s = lax.axis_index("s")
    r0 = s * _RW
    pltpu.sync_copy(zeros_hbm.at[pl.ds(r0, _RW)], deg_sh.at[pl.ds(r0, _RW)])
    pltpu.sync_copy(ones_hbm, ones_v)
    plsc.subcore_barrier()
    base = (c * _NS + s) * (_DEG_CH * _CHUNK)

    def chunk(k, carry):
        b = base + k * _CHUNK
        pltpu.sync_copy(dst_hbm.at[pl.ds(b, _CHUNK)], idx_v)
        pltpu.sync_copy(ones_v, deg_sh.at[idx_v], add=True)
        return carry

    lax.fori_loop(0, _DEG_CH, chunk, 0)
    plsc.subcore_barrier()
    pltpu.sync_copy(deg_sh.at[pl.ds(r0, _RW)],
                    out_hbm.at[pl.ds(c * _N + r0, _RW)])


def _make_prop(n_slices):
    spc = n_slices // _NC

    @functools.partial(
        pl.kernel,
        out_type=jax.ShapeDtypeStruct((n_slices * _N, 128), jnp.float32),
        mesh=_MESH,
        scratch_types=[
            pltpu.VMEM((_CHUNK,), jnp.int32),
            pltpu.VMEM((_CHUNK,), jnp.int32),
            pltpu.VMEM((_CHUNK,), jnp.int32),
            pltpu.VMEM((_CHUNK, 128), jnp.float32),
            pltpu.VMEM_SHARED((_NPAD, 128), jnp.float32),
        ],
    )
    def prop(src_hbm, dst_hbm, z_hbm, out_hbm, srcv, dstv, adjv, rows_v, acc_sh):
        c = lax.axis_index("c")
        s = lax.axis_index("s")
        r0 = s * _RW
        for j in range(spc):
            q = c * spc + j
            qn = q * _N
            pltpu.sync_copy(z_hbm.at[pl.ds(qn + r0, _RW)],
                            acc_sh.at[pl.ds(r0, _RW)])
            plsc.subcore_barrier()

            def chunk(k, carry):
                b = s * (_PROP_CH * _CHUNK) + k * _CHUNK
                pltpu.sync_copy(src_hbm.at[pl.ds(b, _CHUNK)], srcv)
                pltpu.sync_copy(dst_hbm.at[pl.ds(b, _CHUNK)], dstv)
                qnv = jnp.full((16,), qn, jnp.int32)
                for i in range(_CHUNK // 16):
                    adjv[pl.ds(i * 16, 16)] = srcv[pl.ds(i * 16, 16)] + qnv
                pltpu.sync_copy(z_hbm.at[adjv], rows_v)
                pltpu.sync_copy(rows_v, acc_sh.at[dstv], add=True)
                return carry

            lax.fori_loop(0, _PROP_CH, chunk, 0)
            plsc.subcore_barrier()
            pltpu.sync_copy(acc_sh.at[pl.ds(r0, _RW)],
                            out_hbm.at[pl.ds(qn + r0, _RW)])
            plsc.subcore_barrier()

    return prop


_prop4 = _make_prop(4)
_prop2 = _make_prop(2)


def _dinv_block(deg_ref):
    d = deg_ref[0, :, 0] + deg_ref[1, :, 0] + 1.0
    return lax.rsqrt(d)


def _pre1_body(x_ref, deg_ref, we_ref, be_ref, wc_ref, out_ref):
    h0 = jnp.dot(x_ref[...], we_ref[...],
                 preferred_element_type=jnp.float32) + be_ref[...]
    z = jnp.dot(h0, wc_ref[...], preferred_element_type=jnp.float32)
    z = z * _dinv_block(deg_ref)[:, None]
    for q in range(4):
        out_ref[q] = z[:, q * 128:(q + 1) * 128]


def _pre1(x, deg3, w_enc, b_enc, wc1):
    return pl.pallas_call(
        _pre1_body,
        grid=(_G,),
        in_specs=[
            pl.BlockSpec((_R, _IN), lambda i: (i, 0)),
            pl.BlockSpec((2, _R, 16), lambda i: (0, i, 0)),
            pl.BlockSpec((_IN, _H), lambda i: (0, 0)),
            pl.BlockSpec((1, _H), lambda i: (0, 0)),
            pl.BlockSpec((_H, _H), lambda i: (0, 0)),
        ],
        out_specs=pl.BlockSpec((4, _R, 128), lambda i: (0, i, 0)),
        out_shape=jax.ShapeDtypeStruct((4, _N, 128), jnp.float32),
    )(x, deg3, w_enc, b_enc, wc1)


def _make_post(n_slices):
    hs = n_slices * 128

    def body(w_ref, deg_ref, b_ref, u_ref, stats_ref, sum_ref, sq_ref):
        i = pl.program_id(0)
        u = jnp.concatenate([w_ref[q] for q in range(n_slices)], axis=1)
        u = u * _dinv_block(deg_ref)[:, None] + b_ref[...]
        u_ref[...] = u

        @pl.when(i == 0)
        def _():
            sum_ref[...] = jnp.zeros_like(sum_ref)
            sq_ref[...] = jnp.zeros_like(sq_ref)

        sum_ref[...] = sum_ref[...] + jnp.sum(u, axis=0, keepdims=True)
        sq_ref[...] = sq_ref[...] + jnp.sum(u * u, axis=0, keepdims=True)

        @pl.when(i == _G - 1)
        def _():
            stats_ref[0:1, :] = sum_ref[...]
            stats_ref[1:2, :] = sq_ref[...]

    def post(w3d, deg3, b):
        return pl.pallas_call(
            body,
            grid=(_G,),
            in_specs=[
                pl.BlockSpec((n_slices, _R, 128), lambda i: (0, i, 0)),
                pl.BlockSpec((2, _R, 16), lambda i: (0, i, 0)),
                pl.BlockSpec((1, hs), lambda i: (0, 0)),
            ],
            out_specs=[
                pl.BlockSpec((_R, hs), lambda i: (i, 0)),
                pl.BlockSpec((2, hs), lambda i: (0, 0)),
            ],
            out_shape=[
                jax.ShapeDtypeStruct((_N, hs), jnp.float32),
                jax.ShapeDtypeStruct((2, hs), jnp.float32),
            ],
            scratch_shapes=[
                pltpu.VMEM((1, hs), jnp.float32),
                pltpu.VMEM((1, hs), jnp.float32),
            ],
        )(w3d, deg3, b)

    return post


_post4 = _make_post(4)
_post2 = _make_post(2)


def _make_pre(h_in, n_slices):
    def body(u_ref, st_ref, g_ref, bt_ref, deg_ref, w_ref, out_ref):
        st = st_ref[...]
        m = st[0:1, :] * (1.0 / _N)
        var = st[1:2, :] * (1.0 / _N) - m * m
        sc = g_ref[...] * lax.rsqrt(var + 1e-5)
        h = jnp.maximum((u_ref[...] - m) * sc + bt_ref[...], 0.0)
        z = jnp.dot(h, w_ref[...], preferred_element_type=jnp.float32)
        z = z * _dinv_block(deg_ref)[:, None]
        for q in range(n_slices):
            out_ref[q] = z[:, q * 128:(q + 1) * 128]

    def pre(u, st, g, bt, deg3, w):
        return pl.pallas_call(
            body,
            grid=(_G,),
            in_specs=[
                pl.BlockSpec((_R, h_in), lambda i: (i, 0)),
                pl.BlockSpec((2, h_in), lambda i: (0, 0)),
                pl.BlockSpec((1, h_in), lambda i: (0, 0)),
                pl.BlockSpec((1, h_in), lambda i: (0, 0)),
                pl.BlockSpec((2, _R, 16), lambda i: (0, i, 0)),
                pl.BlockSpec((h_in, n_slices * 128), lambda i: (0, 0)),
            ],
            out_specs=pl.BlockSpec((n_slices, _R, 128), lambda i: (0, i, 0)),
            out_shape=jax.ShapeDtypeStruct((n_slices, _N, 128), jnp.float32),
        )(u, st, g, bt, deg3, w)

    return pre


_pre2 = _make_pre(_H, 4)
_pre3 = _make_pre(_H, 2)


def _final_body(u_ref, st_ref, g_ref, bt_ref,
                we1_ref, be1_ref, we2t_ref, be2_ref,
                wm1_ref, bm1_ref, wm2t_ref, bm2_ref,
                wf1_ref, bf1_ref, wf2_ref, bf2_ref,
                emb_ref, ge_ref, e_ref, m_ref, f_ref, csum_ref):
    i = pl.program_id(0)
    st = st_ref[...]
    mean = st[0:1, :] * (1.0 / _N)
    var = st[1:2, :] * (1.0 / _N) - mean * mean
    sc = g_ref[...] * lax.rsqrt(var + 1e-5)
    h3 = (u_ref[...] - mean) * sc + bt_ref[...]
    emb_ref[...] = h3

    @pl.when(i == 0)
    def _():
        csum_ref[...] = jnp.zeros_like(csum_ref)

    csum_ref[...] = csum_ref[...] + jnp.sum(h3, axis=0, keepdims=True)

    @pl.when(i == _G - 1)
    def _():
        ge = csum_ref[...] * (1.0 / _N)
        ge_ref[...] = ge
        ae = jnp.maximum(jnp.dot(ge, we1_ref[...],
                                 preferred_element_type=jnp.float32)
                         + be1_ref[...], 0.0)
        e_ref[...] = jax.nn.sigmoid(
            jnp.sum(ae * we2t_ref[...], axis=1, keepdims=True) + be2_ref[...])
        am = jnp.maximum(jnp.dot(ge, wm1_ref[...],
                                 preferred_element_type=jnp.float32)
                         + bm1_ref[...], 0.0)
        m_ref[...] = jax.nn.sigmoid(
            jnp.sum(am * wm2t_ref[...], axis=1, keepdims=True) + bm2_ref[...])
        af = jnp.maximum(jnp.dot(ge, wf1_ref[...],
                                 preferred_element_type=jnp.float32)
                         + bf1_ref[...], 0.0)
        f_ref[...] = jnp.dot(af, wf2_ref[...],
                             preferred_element_type=jnp.float32) + bf2_ref[...]


def _final(u3, st3, g3, bt3, we1, be1, we2t, be2, wm1, bm1, wm2t, bm2,
           wf1, bf1, wf2, bf2):
    hh = _OUT // 2

    def full(shape):
        return pl.BlockSpec(shape, lambda i: tuple(0 for _ in shape))

    return pl.pallas_call(
        _final_body,
        grid=(_G,),
        in_specs=[
            pl.BlockSpec((_R, _OUT), lambda i: (i, 0)),
            full((2, _OUT)), full((1, _OUT)), full((1, _OUT)),
            full((_OUT, hh)), full((1, hh)), full((1, hh)), full((1, 1)),
            full((_OUT, hh)), full((1, hh)), full((1, hh)), full((1, 1)),
            full((_OUT, hh)), full((1, hh)), full((hh, 6)), full((1, 6)),
        ],
        out_specs=[
            pl.BlockSpec((_R, _OUT), lambda i: (i, 0)),
            full((1, _OUT)), full((1, 1)), full((1, 1)), full((1, 6)),
        ],
        out_shape=[
            jax.ShapeDtypeStruct((_N, _OUT), jnp.float32),
            jax.ShapeDtypeStruct((1, _OUT), jnp.float32),
            jax.ShapeDtypeStruct((1, 1), jnp.float32),
            jax.ShapeDtypeStruct((1, 1), jnp.float32),
            jax.ShapeDtypeStruct((1, 6), jnp.float32),
        ],
        scratch_shapes=[pltpu.VMEM((1, _OUT), jnp.float32)],
    )(u3, st3, g3, bt3, we1, be1, we2t, be2, wm1, bm1, wm2t, bm2,
      wf1, bf1, wf2, bf2)


def kernel(x, edge_index, W_enc, b_enc, Wc1, bc1, g1, beta1, Wc2, bc2, g2,
           beta2, Wc3, bc3, g3, beta3, We1, be1, We2, be2, Wm1, bm1, Wm2,
           bm2, Wf1, bf1, Wf2, bf2):
    pad = _EPAD - _E
    src_p = jnp.concatenate([edge_index[0], jnp.zeros((pad,), jnp.int32)])
    dst_p = jnp.concatenate([edge_index[1], jnp.full((pad,), _N, jnp.int32)])
    zeros16 = jnp.zeros((_N, 16), jnp.float32)
    ones16 = jnp.ones((_CHUNK, 16), jnp.float32)

    def row(v):
        return v.reshape(1, -1)

    deg2 = _deg_kernel(dst_p, zeros16, ones16)
    deg3 = deg2.reshape(2, _N, 16)

    z1 = _pre1(x, deg3, W_enc, row(b_enc), Wc1)
    w1 = _prop4(src_p, dst_p, z1.reshape(4 * _N, 128))
    u1, st1 = _post4(w1.reshape(4, _N, 128), deg3, row(bc1))

    z2 = _pre2(u1, st1, row(g1), row(beta1), deg3, Wc2)
    w2 = _prop4(src_p, dst_p, z2.reshape(4 * _N, 128))
    u2, st2 = _post4(w2.reshape(4, _N, 128), deg3, row(bc2))

    z3 = _pre3(u2, st2, row(g2), row(beta2), deg3, Wc3)
    w3 = _prop2(src_p, dst_p, z3.reshape(2 * _N, 128))
    u3, st3 = _post2(w3.reshape(2, _N, 128), deg3, row(bc3))

    emb, ge, e, m, f = _final(
        u3, st3, row(g3), row(beta3),
        We1, row(be1), We2.reshape(1, -1), row(be2),
        Wm1, row(bm1), Wm2.reshape(1, -1), row(bm2),
        Wf1, row(bf1), Wf2, row(bf2))
    return (emb, ge, e, m, f)


# speedup vs baseline: 4.4806x; 4.4806x over previous
import functools

import jax
import jax.numpy as jnp
from jax import lax
from jax.experimental import pallas as pl
from jax.experimental.pallas import tpu as pltpu
from jax.experimental.pallas import tpu_sc as plsc

_N = 10000
_E = 160000
_IN = 256
_H = 512
_OUT = 256

_R = 1000
_G = _N // _R

_NS = 16
_NC = 2
_CHUNK = 128
_PROP_CH = 80
_EPAD = _NS * _PROP_CH * _CHUNK
_DEG_CH = _EPAD // (_NC * _NS * _CHUNK)
_RB = 624
_RL = 640
_NPAD = 10016


def _mesh():
    return plsc.VectorSubcoreMesh(core_axis_name="c", subcore_axis_name="s")


@functools.cache
def _deg():
    @functools.partial(
        pl.kernel,
        out_type=jax.ShapeDtypeStruct((_NC * _N, 128), jnp.float32),
        mesh=_mesh(),
        scratch_types=[
            pltpu.VMEM((_CHUNK,), jnp.int32),
            pltpu.VMEM((_CHUNK, 128), jnp.float32),
            pltpu.VMEM_SHARED((_NPAD, 128), jnp.float32),
        ],
    )
    def deg_kernel(dst_hbm, zeros_hbm, ones_hbm, out_hbm, idx_v, ones_v, deg_sh):
        c = lax.axis_index("c")
        s = lax.axis_index("s")
        r0 = s * _RB
        pltpu.sync_copy(zeros_hbm, deg_sh.at[pl.ds(r0, _RL)])
        pltpu.sync_copy(ones_hbm, ones_v)
        plsc.subcore_barrier()
        base = (c * _NS + s) * (_DEG_CH * _CHUNK)

        def chunk(k, carry):
            b = base + k * _CHUNK
            pltpu.sync_copy(dst_hbm.at[pl.ds(b, _CHUNK)], idx_v)
            pltpu.sync_copy(ones_v, deg_sh.at[idx_v], add=True)
            return carry

        lax.fori_loop(0, _DEG_CH, chunk, 0)
        plsc.subcore_barrier()
        pltpu.sync_copy(deg_sh.at[pl.ds(r0, _RL)],
                        out_hbm.at[pl.ds(c * _N + r0, _RL)])

    return deg_kernel


@functools.cache
def _prop(n_slices):
    spc = n_slices // _NC

    @functools.partial(
        pl.kernel,
        out_type=jax.ShapeDtypeStruct((n_slices * _N, 128), jnp.float32),
        mesh=_mesh(),
        scratch_types=[
            pltpu.VMEM((_CHUNK,), jnp.int32),
            pltpu.VMEM((_CHUNK,), jnp.int32),
            pltpu.VMEM((_CHUNK,), jnp.int32),
            pltpu.VMEM((_CHUNK, 128), jnp.float32),
            pltpu.VMEM_SHARED((_NPAD, 128), jnp.float32),
        ],
    )
    def prop(src_hbm, dst_hbm, z_hbm, out_hbm, srcv, dstv, adjv, rows_v, acc_sh):
        c = lax.axis_index("c")
        s = lax.axis_index("s")
        r0 = s * _RB
        for j in range(spc):
            q = c * spc + j
            qn = q * _N
            pltpu.sync_copy(z_hbm.at[pl.ds(qn + r0, _RL)],
                            acc_sh.at[pl.ds(r0, _RL)])
            plsc.subcore_barrier()

            def chunk(k, carry):
                b = s * (_PROP_CH * _CHUNK) + k * _CHUNK
                pltpu.sync_copy(src_hbm.at[pl.ds(b, _CHUNK)], srcv)
                pltpu.sync_copy(dst_hbm.at[pl.ds(b, _CHUNK)], dstv)
                qnv = jnp.full((16,), qn, jnp.int32)
                for i in range(_CHUNK // 16):
                    adjv[pl.ds(i * 16, 16)] = srcv[pl.ds(i * 16, 16)] + qnv
                pltpu.sync_copy(z_hbm.at[adjv], rows_v)
                pltpu.sync_copy(rows_v, acc_sh.at[dstv], add=True)
                return carry

            lax.fori_loop(0, _PROP_CH, chunk, 0)
            plsc.subcore_barrier()
            pltpu.sync_copy(acc_sh.at[pl.ds(r0, _RL)],
                            out_hbm.at[pl.ds(qn + r0, _RL)])
            plsc.subcore_barrier()

    return prop


def _dinv_block(deg_ref):
    d = deg_ref[0, :, 0] + deg_ref[1, :, 0] + 1.0
    return lax.rsqrt(d)


def _pre1_body(x_ref, deg_ref, we_ref, be_ref, wc_ref, out_ref):
    h0 = jnp.dot(x_ref[...], we_ref[...],
                 preferred_element_type=jnp.float32) + be_ref[...]
    z = jnp.dot(h0, wc_ref[...], preferred_element_type=jnp.float32)
    z = z * _dinv_block(deg_ref)[:, None]
    for q in range(4):
        out_ref[q] = z[:, q * 128:(q + 1) * 128]


def _pre1(x, deg3, w_enc, b_enc, wc1):
    return pl.pallas_call(
        _pre1_body,
        grid=(_G,),
        in_specs=[
            pl.BlockSpec((_R, _IN), lambda i: (i, 0)),
            pl.BlockSpec((2, _R, 128), lambda i: (0, i, 0)),
            pl.BlockSpec((_IN, _H), lambda i: (0, 0)),
            pl.BlockSpec((1, _H), lambda i: (0, 0)),
            pl.BlockSpec((_H, _H), lambda i: (0, 0)),
        ],
        out_specs=pl.BlockSpec((4, _R, 128), lambda i: (0, i, 0)),
        out_shape=jax.ShapeDtypeStruct((4, _N, 128), jnp.float32),
    )(x, deg3, w_enc, b_enc, wc1)


def _make_post(n_slices):
    hs = n_slices * 128

    def body(w_ref, deg_ref, b_ref, u_ref, stats_ref, sum_ref, sq_ref):
        i = pl.program_id(0)
        u = jnp.concatenate([w_ref[q] for q in range(n_slices)], axis=1)
        u = u * _dinv_block(deg_ref)[:, None] + b_ref[...]
        u_ref[...] = u

        @pl.when(i == 0)
        def _():
            sum_ref[...] = jnp.zeros_like(sum_ref)
            sq_ref[...] = jnp.zeros_like(sq_ref)

        sum_ref[...] = sum_ref[...] + jnp.sum(u, axis=0, keepdims=True)
        sq_ref[...] = sq_ref[...] + jnp.sum(u * u, axis=0, keepdims=True)

        @pl.when(i == _G - 1)
        def _():
            stats_ref[0:1, :] = sum_ref[...]
            stats_ref[1:2, :] = sq_ref[...]

    def post(w3d, deg3, b):
        return pl.pallas_call(
            body,
            grid=(_G,),
            in_specs=[
                pl.BlockSpec((n_slices, _R, 128), lambda i: (0, i, 0)),
                pl.BlockSpec((2, _R, 128), lambda i: (0, i, 0)),
                pl.BlockSpec((1, hs), lambda i: (0, 0)),
            ],
            out_specs=[
                pl.BlockSpec((_R, hs), lambda i: (i, 0)),
                pl.BlockSpec((2, hs), lambda i: (0, 0)),
            ],
            out_shape=[
                jax.ShapeDtypeStruct((_N, hs), jnp.float32),
                jax.ShapeDtypeStruct((2, hs), jnp.float32),
            ],
            scratch_shapes=[
                pltpu.VMEM((1, hs), jnp.float32),
                pltpu.VMEM((1, hs), jnp.float32),
            ],
        )(w3d, deg3, b)

    return post


_post4 = _make_post(4)
_post2 = _make_post(2)


def _make_pre(h_in, n_slices):
    def body(u_ref, st_ref, g_ref, bt_ref, deg_ref, w_ref, out_ref):
        st = st_ref[...]
        m = st[0:1, :] * (1.0 / _N)
        var = st[1:2, :] * (1.0 / _N) - m * m
        sc = g_ref[...] * lax.rsqrt(var + 1e-5)
        h = jnp.maximum((u_ref[...] - m) * sc + bt_ref[...], 0.0)
        z = jnp.dot(h, w_ref[...], preferred_element_type=jnp.float32)
        z = z * _dinv_block(deg_ref)[:, None]
        for q in range(n_slices):
            out_ref[q] = z[:, q * 128:(q + 1) * 128]

    def pre(u, st, g, bt, deg3, w):
        return pl.pallas_call(
            body,
            grid=(_G,),
            in_specs=[
                pl.BlockSpec((_R, h_in), lambda i: (i, 0)),
                pl.BlockSpec((2, h_in), lambda i: (0, 0)),
                pl.BlockSpec((1, h_in), lambda i: (0, 0)),
                pl.BlockSpec((1, h_in), lambda i: (0, 0)),
                pl.BlockSpec((2, _R, 128), lambda i: (0, i, 0)),
                pl.BlockSpec((h_in, n_slices * 128), lambda i: (0, 0)),
            ],
            out_specs=pl.BlockSpec((n_slices, _R, 128), lambda i: (0, i, 0)),
            out_shape=jax.ShapeDtypeStruct((n_slices, _N, 128), jnp.float32),
        )(u, st, g, bt, deg3, w)

    return pre


_pre2 = _make_pre(_H, 4)
_pre3 = _make_pre(_H, 2)


def _final_body(u_ref, st_ref, g_ref, bt_ref,
                we1_ref, be1_ref, we2t_ref, be2_ref,
                wm1_ref, bm1_ref, wm2t_ref, bm2_ref,
                wf1_ref, bf1_ref, wf2_ref, bf2_ref,
                emb_ref, ge_ref, e_ref, m_ref, f_ref, csum_ref):
    i = pl.program_id(0)
    st = st_ref[...]
    mean = st[0:1, :] * (1.0 / _N)
    var = st[1:2, :] * (1.0 / _N) - mean * mean
    sc = g_ref[...] * lax.rsqrt(var + 1e-5)
    h3 = (u_ref[...] - mean) * sc + bt_ref[...]
    emb_ref[...] = h3

    @pl.when(i == 0)
    def _():
        csum_ref[...] = jnp.zeros_like(csum_ref)

    csum_ref[...] = csum_ref[...] + jnp.sum(h3, axis=0, keepdims=True)

    @pl.when(i == _G - 1)
    def _():
        ge = csum_ref[...] * (1.0 / _N)
        ge_ref[...] = ge
        ae = jnp.maximum(jnp.dot(ge, we1_ref[...],
                                 preferred_element_type=jnp.float32)
                         + be1_ref[...], 0.0)
        e_ref[...] = jax.nn.sigmoid(
            jnp.sum(ae * we2t_ref[...], axis=1, keepdims=True) + be2_ref[...])
        am = jnp.maximum(jnp.dot(ge, wm1_ref[...],
                                 preferred_element_type=jnp.float32)
                         + bm1_ref[...], 0.0)
        m_ref[...] = jax.nn.sigmoid(
            jnp.sum(am * wm2t_ref[...], axis=1, keepdims=True) + bm2_ref[...])
        af = jnp.maximum(jnp.dot(ge, wf1_ref[...],
                                 preferred_element_type=jnp.float32)
                         + bf1_ref[...], 0.0)
        f_ref[...] = jnp.dot(af, wf2_ref[...],
                             preferred_element_type=jnp.float32) + bf2_ref[...]


def _final(u3, st3, g3, bt3, we1, be1, we2t, be2, wm1, bm1, wm2t, bm2,
           wf1, bf1, wf2, bf2):
    hh = _OUT // 2

    def full(shape):
        return pl.BlockSpec(shape, lambda i: tuple(0 for _ in shape))

    return pl.pallas_call(
        _final_body,
        grid=(_G,),
        in_specs=[
            pl.BlockSpec((_R, _OUT), lambda i: (i, 0)),
            full((2, _OUT)), full((1, _OUT)), full((1, _OUT)),
            full((_OUT, hh)), full((1, hh)), full((1, hh)), full((1, 1)),
            full((_OUT, hh)), full((1, hh)), full((1, hh)), full((1, 1)),
            full((_OUT, hh)), full((1, hh)), full((hh, 6)), full((1, 6)),
        ],
        out_specs=[
            pl.BlockSpec((_R, _OUT), lambda i: (i, 0)),
            full((1, _OUT)), full((1, 1)), full((1, 1)), full((1, 6)),
        ],
        out_shape=[
            jax.ShapeDtypeStruct((_N, _OUT), jnp.float32),
            jax.ShapeDtypeStruct((1, _OUT), jnp.float32),
            jax.ShapeDtypeStruct((1, 1), jnp.float32),
            jax.ShapeDtypeStruct((1, 1), jnp.float32),
            jax.ShapeDtypeStruct((1, 6), jnp.float32),
        ],
        scratch_shapes=[pltpu.VMEM((1, _OUT), jnp.float32)],
    )(u3, st3, g3, bt3, we1, be1, we2t, be2, wm1, bm1, wm2t, bm2,
      wf1, bf1, wf2, bf2)


def kernel(x, edge_index, W_enc, b_enc, Wc1, bc1, g1, beta1, Wc2, bc2, g2,
           beta2, Wc3, bc3, g3, beta3, We1, be1, We2, be2, Wm1, bm1, Wm2,
           bm2, Wf1, bf1, Wf2, bf2):
    pad = _EPAD - _E
    src_p = jnp.concatenate([edge_index[0], jnp.zeros((pad,), jnp.int32)])
    dst_p = jnp.concatenate([edge_index[1], jnp.full((pad,), _N, jnp.int32)])
    zeros0 = jnp.zeros((_RL, 128), jnp.float32)
    ones0 = jnp.ones((_CHUNK, 128), jnp.float32)

    def row(v):
        return v.reshape(1, -1)

    deg2 = _deg()(dst_p, zeros0, ones0)
    deg3 = deg2.reshape(2, _N, 128)

    z1 = _pre1(x, deg3, W_enc, row(b_enc), Wc1)
    w1 = _prop(4)(src_p, dst_p, z1.reshape(4 * _N, 128))
    u1, st1 = _post4(w1.reshape(4, _N, 128), deg3, row(bc1))

    z2 = _pre2(u1, st1, row(g1), row(beta1), deg3, Wc2)
    w2 = _prop(4)(src_p, dst_p, z2.reshape(4 * _N, 128))
    u2, st2 = _post4(w2.reshape(4, _N, 128), deg3, row(bc2))

    z3 = _pre3(u2, st2, row(g2), row(beta2), deg3, Wc3)
    w3 = _prop(2)(src_p, dst_p, z3.reshape(2 * _N, 128))
    u3, st3 = _post2(w3.reshape(2, _N, 128), deg3, row(bc3))

    emb, ge, e, m, f = _final(
        u3, st3, row(g3), row(beta3),
        We1, row(be1), We2.reshape(1, -1), row(be2),
        Wm1, row(bm1), Wm2.reshape(1, -1), row(bm2),
        Wf1, row(bf1), Wf2, row(bf2))
    return (emb, ge, e, m, f)


# speedup vs baseline: 5.3458x; 1.1931x over previous
import functools

import jax
import jax.numpy as jnp
from jax import lax
from jax.experimental import pallas as pl
from jax.experimental.pallas import tpu as pltpu
from jax.experimental.pallas import tpu_sc as plsc

_N = 10000
_E = 160000
_IN = 256
_H = 512
_OUT = 256

_R = 1000
_G = _N // _R

_NS = 16
_NC = 2
_CHUNK = 128
_PCHUNK = 64
_PROP_CH = 160
_EPAD = _NS * _PROP_CH * _PCHUNK
_DEG_CH = _EPAD // (_NC * _NS * _CHUNK)
_RB = 624
_RL = 640
_NPAD = 10016


def _mesh():
    return plsc.VectorSubcoreMesh(core_axis_name="c", subcore_axis_name="s")


@functools.cache
def _deg():
    @functools.partial(
        pl.kernel,
        out_type=jax.ShapeDtypeStruct((_NC * _N, 128), jnp.float32),
        mesh=_mesh(),
        scratch_types=[
            pltpu.VMEM((_CHUNK,), jnp.int32),
            pltpu.VMEM((_CHUNK, 128), jnp.float32),
            pltpu.VMEM_SHARED((_NPAD, 128), jnp.float32),
        ],
    )
    def deg_kernel(dst_hbm, zeros_hbm, ones_hbm, out_hbm, idx_v, ones_v, deg_sh):
        c = lax.axis_index("c")
        s = lax.axis_index("s")
        r0 = s * _RB
        pltpu.sync_copy(zeros_hbm, deg_sh.at[pl.ds(r0, _RL)])
        pltpu.sync_copy(ones_hbm, ones_v)
        plsc.subcore_barrier()
        base = (c * _NS + s) * (_DEG_CH * _CHUNK)

        def chunk(k, carry):
            b = base + k * _CHUNK
            pltpu.sync_copy(dst_hbm.at[pl.ds(b, _CHUNK)], idx_v)
            pltpu.sync_copy(ones_v, deg_sh.at[idx_v], add=True)
            return carry

        lax.fori_loop(0, _DEG_CH, chunk, 0)
        plsc.subcore_barrier()
        pltpu.sync_copy(deg_sh.at[pl.ds(r0, _RL)],
                        out_hbm.at[pl.ds(c * _N + r0, _RL)])

    return deg_kernel


_NBUF = 4
_HALF = _PROP_CH // 4


@functools.cache
def _prop(n_slices):
    spc = n_slices // _NC

    @functools.partial(
        pl.kernel,
        out_type=jax.ShapeDtypeStruct((n_slices * _N, 128), jnp.float32),
        mesh=_mesh(),
        scratch_types=[
            pltpu.VMEM((_HALF, _PCHUNK), jnp.int32),
            pltpu.VMEM((_HALF, _PCHUNK), jnp.int32),
            pltpu.VMEM((_NBUF, _PCHUNK, 128), jnp.float32),
            pltpu.VMEM_SHARED((_NPAD, 128), jnp.float32),
        ] + [pltpu.SemaphoreType.DMA] * _NBUF,
    )
    def prop(srcadj_hbm, dst_hbm, z_hbm, out_hbm, srcv2, dstv2, rows2,
             acc_sh, *sems):
        c = lax.axis_index("c")
        s = lax.axis_index("s")
        r0 = s * _RB
        for j in range(spc):
            q = c * spc + j
            pltpu.sync_copy(z_hbm.at[pl.ds(q * _N + r0, _RL)],
                            acc_sh.at[pl.ds(r0, _RL)])
            plsc.subcore_barrier()

            for half in range(4):
                ch0 = half * _HALF
                pltpu.sync_copy(
                    srcadj_hbm.at[pl.ds((q * _NS + s) * _PROP_CH + ch0,
                                        _HALF)], srcv2)
                pltpu.sync_copy(
                    dst_hbm.at[pl.ds(s * _PROP_CH + ch0, _HALF)], dstv2)

                def duo(t, carry):
                    k0 = t * _NBUF
                    descs = [
                        pltpu.async_copy(z_hbm.at[srcv2.at[k0 + b]],
                                         rows2.at[b], sems[b])
                        for b in range(_NBUF)
                    ]
                    for b in range(_NBUF):
                        descs[b].wait()
                        pltpu.sync_copy(rows2.at[b],
                                        acc_sh.at[dstv2.at[k0 + b]],
                                        add=True)
                    return carry

                lax.fori_loop(0, _HALF // _NBUF, duo, 0)

            plsc.subcore_barrier()
            pltpu.sync_copy(acc_sh.at[pl.ds(r0, _RL)],
                            out_hbm.at[pl.ds(q * _N + r0, _RL)])
            plsc.subcore_barrier()

    return prop


def _dinv_block(deg_ref):
    d = deg_ref[0, :, 0] + deg_ref[1, :, 0] + 1.0
    return lax.rsqrt(d)


def _pre1_body(x_ref, deg_ref, we_ref, be_ref, wc_ref, out_ref):
    h0 = jnp.dot(x_ref[...], we_ref[...],
                 preferred_element_type=jnp.float32) + be_ref[...]
    z = jnp.dot(h0, wc_ref[...], preferred_element_type=jnp.float32)
    z = z * _dinv_block(deg_ref)[:, None]
    for q in range(4):
        out_ref[q] = z[:, q * 128:(q + 1) * 128]


def _pre1(x, deg3, w_enc, b_enc, wc1):
    return pl.pallas_call(
        _pre1_body,
        grid=(_G,),
        in_specs=[
            pl.BlockSpec((_R, _IN), lambda i: (i, 0)),
            pl.BlockSpec((2, _R, 128), lambda i: (0, i, 0)),
            pl.BlockSpec((_IN, _H), lambda i: (0, 0)),
            pl.BlockSpec((1, _H), lambda i: (0, 0)),
            pl.BlockSpec((_H, _H), lambda i: (0, 0)),
        ],
        out_specs=pl.BlockSpec((4, _R, 128), lambda i: (0, i, 0)),
        out_shape=jax.ShapeDtypeStruct((4, _N, 128), jnp.float32),
    )(x, deg3, w_enc, b_enc, wc1)


def _make_post(n_slices):
    hs = n_slices * 128

    def body(w_ref, deg_ref, b_ref, u_ref, stats_ref, sum_ref, sq_ref):
        i = pl.program_id(0)
        u = jnp.concatenate([w_ref[q] for q in range(n_slices)], axis=1)
        u = u * _dinv_block(deg_ref)[:, None] + b_ref[...]
        u_ref[...] = u

        @pl.when(i == 0)
        def _():
            sum_ref[...] = jnp.zeros_like(sum_ref)
            sq_ref[...] = jnp.zeros_like(sq_ref)

        sum_ref[...] = sum_ref[...] + jnp.sum(u, axis=0, keepdims=True)
        sq_ref[...] = sq_ref[...] + jnp.sum(u * u, axis=0, keepdims=True)

        @pl.when(i == _G - 1)
        def _():
            stats_ref[0:1, :] = sum_ref[...]
            stats_ref[1:2, :] = sq_ref[...]

    def post(w3d, deg3, b):
        return pl.pallas_call(
            body,
            grid=(_G,),
            in_specs=[
                pl.BlockSpec((n_slices, _R, 128), lambda i: (0, i, 0)),
                pl.BlockSpec((2, _R, 128), lambda i: (0, i, 0)),
                pl.BlockSpec((1, hs), lambda i: (0, 0)),
            ],
            out_specs=[
                pl.BlockSpec((_R, hs), lambda i: (i, 0)),
                pl.BlockSpec((2, hs), lambda i: (0, 0)),
            ],
            out_shape=[
                jax.ShapeDtypeStruct((_N, hs), jnp.float32),
                jax.ShapeDtypeStruct((2, hs), jnp.float32),
            ],
            scratch_shapes=[
                pltpu.VMEM((1, hs), jnp.float32),
                pltpu.VMEM((1, hs), jnp.float32),
            ],
        )(w3d, deg3, b)

    return post


_post4 = _make_post(4)
_post2 = _make_post(2)


def _make_pre(h_in, n_slices):
    def body(u_ref, st_ref, g_ref, bt_ref, deg_ref, w_ref, out_ref):
        st = st_ref[...]
        m = st[0:1, :] * (1.0 / _N)
        var = st[1:2, :] * (1.0 / _N) - m * m
        sc = g_ref[...] * lax.rsqrt(var + 1e-5)
        h = jnp.maximum((u_ref[...] - m) * sc + bt_ref[...], 0.0)
        z = jnp.dot(h, w_ref[...], preferred_element_type=jnp.float32)
        z = z * _dinv_block(deg_ref)[:, None]
        for q in range(n_slices):
            out_ref[q] = z[:, q * 128:(q + 1) * 128]

    def pre(u, st, g, bt, deg3, w):
        return pl.pallas_call(
            body,
            grid=(_G,),
            in_specs=[
                pl.BlockSpec((_R, h_in), lambda i: (i, 0)),
                pl.BlockSpec((2, h_in), lambda i: (0, 0)),
                pl.BlockSpec((1, h_in), lambda i: (0, 0)),
                pl.BlockSpec((1, h_in), lambda i: (0, 0)),
                pl.BlockSpec((2, _R, 128), lambda i: (0, i, 0)),
                pl.BlockSpec((h_in, n_slices * 128), lambda i: (0, 0)),
            ],
            out_specs=pl.BlockSpec((n_slices, _R, 128), lambda i: (0, i, 0)),
            out_shape=jax.ShapeDtypeStruct((n_slices, _N, 128), jnp.float32),
        )(u, st, g, bt, deg3, w)

    return pre


_pre2 = _make_pre(_H, 4)
_pre3 = _make_pre(_H, 2)


def _final_body(u_ref, st_ref, g_ref, bt_ref,
                we1_ref, be1_ref, we2t_ref, be2_ref,
                wm1_ref, bm1_ref, wm2t_ref, bm2_ref,
                wf1_ref, bf1_ref, wf2_ref, bf2_ref,
                emb_ref, ge_ref, e_ref, m_ref, f_ref, csum_ref):
    i = pl.program_id(0)
    st = st_ref[...]
    mean = st[0:1, :] * (1.0 / _N)
    var = st[1:2, :] * (1.0 / _N) - mean * mean
    sc = g_ref[...] * lax.rsqrt(var + 1e-5)
    h3 = (u_ref[...] - mean) * sc + bt_ref[...]
    emb_ref[...] = h3

    @pl.when(i == 0)
    def _():
        csum_ref[...] = jnp.zeros_like(csum_ref)

    csum_ref[...] = csum_ref[...] + jnp.sum(h3, axis=0, keepdims=True)

    @pl.when(i == _G - 1)
    def _():
        ge = csum_ref[...] * (1.0 / _N)
        ge_ref[...] = ge
        ae = jnp.maximum(jnp.dot(ge, we1_ref[...],
                                 preferred_element_type=jnp.float32)
                         + be1_ref[...], 0.0)
        e_ref[...] = jax.nn.sigmoid(
            jnp.sum(ae * we2t_ref[...], axis=1, keepdims=True) + be2_ref[...])
        am = jnp.maximum(jnp.dot(ge, wm1_ref[...],
                                 preferred_element_type=jnp.float32)
                         + bm1_ref[...], 0.0)
        m_ref[...] = jax.nn.sigmoid(
            jnp.sum(am * wm2t_ref[...], axis=1, keepdims=True) + bm2_ref[...])
        af = jnp.maximum(jnp.dot(ge, wf1_ref[...],
                                 preferred_element_type=jnp.float32)
                         + bf1_ref[...], 0.0)
        f_ref[...] = jnp.dot(af, wf2_ref[...],
                             preferred_element_type=jnp.float32) + bf2_ref[...]


def _final(u3, st3, g3, bt3, we1, be1, we2t, be2, wm1, bm1, wm2t, bm2,
           wf1, bf1, wf2, bf2):
    hh = _OUT // 2

    def full(shape):
        return pl.BlockSpec(shape, lambda i: tuple(0 for _ in shape))

    return pl.pallas_call(
        _final_body,
        grid=(_G,),
        in_specs=[
            pl.BlockSpec((_R, _OUT), lambda i: (i, 0)),
            full((2, _OUT)), full((1, _OUT)), full((1, _OUT)),
            full((_OUT, hh)), full((1, hh)), full((1, hh)), full((1, 1)),
            full((_OUT, hh)), full((1, hh)), full((1, hh)), full((1, 1)),
            full((_OUT, hh)), full((1, hh)), full((hh, 6)), full((1, 6)),
        ],
        out_specs=[
            pl.BlockSpec((_R, _OUT), lambda i: (i, 0)),
            full((1, _OUT)), full((1, 1)), full((1, 1)), full((1, 6)),
        ],
        out_shape=[
            jax.ShapeDtypeStruct((_N, _OUT), jnp.float32),
            jax.ShapeDtypeStruct((1, _OUT), jnp.float32),
            jax.ShapeDtypeStruct((1, 1), jnp.float32),
            jax.ShapeDtypeStruct((1, 1), jnp.float32),
            jax.ShapeDtypeStruct((1, 6), jnp.float32),
        ],
        scratch_shapes=[pltpu.VMEM((1, _OUT), jnp.float32)],
    )(u3, st3, g3, bt3, we1, be1, we2t, be2, wm1, bm1, wm2t, bm2,
      wf1, bf1, wf2, bf2)


def kernel(x, edge_index, W_enc, b_enc, Wc1, bc1, g1, beta1, Wc2, bc2, g2,
           beta2, Wc3, bc3, g3, beta3, We1, be1, We2, be2, Wm1, bm1, Wm2,
           bm2, Wf1, bf1, Wf2, bf2):
    pad = _EPAD - _E
    src_p = jnp.concatenate([edge_index[0], jnp.zeros((pad,), jnp.int32)])
    dst_p = jnp.concatenate([edge_index[1], jnp.full((pad,), _N, jnp.int32)])
    dst2 = dst_p.reshape(_NS * _PROP_CH, _PCHUNK)
    off = jnp.arange(4, dtype=jnp.int32)[:, None] * _N
    srcadj4 = (src_p[None, :] + off).reshape(4 * _NS * _PROP_CH, _PCHUNK)
    srcadj2 = (src_p[None, :] + off[:2]).reshape(2 * _NS * _PROP_CH, _PCHUNK)
    zeros0 = jnp.zeros((_RL, 128), jnp.float32)
    ones0 = jnp.ones((_CHUNK, 128), jnp.float32)

    def row(v):
        return v.reshape(1, -1)

    deg2 = _deg()(dst_p, zeros0, ones0)
    deg3 = deg2.reshape(2, _N, 128)

    z1 = _pre1(x, deg3, W_enc, row(b_enc), Wc1)
    w1 = _prop(4)(srcadj4, dst2, z1.reshape(4 * _N, 128))
    u1, st1 = _post4(w1.reshape(4, _N, 128), deg3, row(bc1))

    z2 = _pre2(u1, st1, row(g1), row(beta1), deg3, Wc2)
    w2 = _prop(4)(srcadj4, dst2, z2.reshape(4 * _N, 128))
    u2, st2 = _post4(w2.reshape(4, _N, 128), deg3, row(bc2))

    z3 = _pre3(u2, st2, row(g2), row(beta2), deg3, Wc3)
    w3 = _prop(2)(srcadj2, dst2, z3.reshape(2 * _N, 128))
    u3, st3 = _post2(w3.reshape(2, _N, 128), deg3, row(bc3))

    emb, ge, e, m, f = _final(
        u3, st3, row(g3), row(beta3),
        We1, row(be1), We2.reshape(1, -1), row(be2),
        Wm1, row(bm1), Wm2.reshape(1, -1), row(bm2),
        Wf1, row(bf1), Wf2, row(bf2))
    return (emb, ge, e, m, f)


# speedup vs baseline: 5.4860x; 1.0262x over previous
import functools

import jax
import jax.numpy as jnp
from jax import lax
from jax.experimental import pallas as pl
from jax.experimental.pallas import tpu as pltpu
from jax.experimental.pallas import tpu_sc as plsc

_N = 10000
_E = 160000
_IN = 256
_H = 512
_OUT = 256

_R = 1000
_G = _N // _R

_NS = 16
_NC = 2
_CHUNK = 128
_PCHUNK = 128
_PROP_CH = 80
_EPAD = _NS * _PROP_CH * _PCHUNK
_DEG_CH = _EPAD // (_NC * _NS * _CHUNK)
_RB = 624
_RL = 640
_NPAD = 10016


def _mesh():
    return plsc.VectorSubcoreMesh(core_axis_name="c", subcore_axis_name="s")


@functools.cache
def _deg():
    @functools.partial(
        pl.kernel,
        out_type=jax.ShapeDtypeStruct((_NC * _N, 128), jnp.float32),
        mesh=_mesh(),
        scratch_types=[
            pltpu.VMEM((_CHUNK,), jnp.int32),
            pltpu.VMEM((_CHUNK, 128), jnp.float32),
            pltpu.VMEM_SHARED((_NPAD, 128), jnp.float32),
        ],
    )
    def deg_kernel(dst_hbm, zeros_hbm, ones_hbm, out_hbm, idx_v, ones_v, deg_sh):
        c = lax.axis_index("c")
        s = lax.axis_index("s")
        r0 = s * _RB
        pltpu.sync_copy(zeros_hbm, deg_sh.at[pl.ds(r0, _RL)])
        pltpu.sync_copy(ones_hbm, ones_v)
        plsc.subcore_barrier()
        base = (c * _NS + s) * (_DEG_CH * _CHUNK)

        def chunk(k, carry):
            b = base + k * _CHUNK
            pltpu.sync_copy(dst_hbm.at[pl.ds(b, _CHUNK)], idx_v)
            pltpu.sync_copy(ones_v, deg_sh.at[idx_v], add=True)
            return carry

        lax.fori_loop(0, _DEG_CH, chunk, 0)
        plsc.subcore_barrier()
        pltpu.sync_copy(deg_sh.at[pl.ds(r0, _RL)],
                        out_hbm.at[pl.ds(c * _N + r0, _RL)])

    return deg_kernel


_NBUF = 2
_HALF = _PROP_CH // 2


@functools.cache
def _prop(n_slices):
    spc = n_slices // _NC

    @functools.partial(
        pl.kernel,
        out_type=jax.ShapeDtypeStruct((n_slices * _N, 128), jnp.float32),
        mesh=_mesh(),
        scratch_types=[
            pltpu.VMEM((_HALF, _PCHUNK), jnp.int32),
            pltpu.VMEM((_HALF, _PCHUNK), jnp.int32),
            pltpu.VMEM((_NBUF, _PCHUNK, 128), jnp.float32),
            pltpu.VMEM_SHARED((_NPAD, 128), jnp.float32),
        ] + [pltpu.SemaphoreType.DMA] * _NBUF,
    )
    def prop(srcadj_hbm, dst_hbm, z_hbm, out_hbm, srcv2, dstv2, rows2,
             acc_sh, *sems):
        c = lax.axis_index("c")
        s = lax.axis_index("s")
        r0 = s * _RB
        for j in range(spc):
            q = c * spc + j
            pltpu.sync_copy(z_hbm.at[pl.ds(q * _N + r0, _RL)],
                            acc_sh.at[pl.ds(r0, _RL)])
            plsc.subcore_barrier()

            for half in range(2):
                ch0 = half * _HALF
                pltpu.sync_copy(
                    srcadj_hbm.at[pl.ds((q * _NS + s) * _PROP_CH + ch0,
                                        _HALF)], srcv2)
                pltpu.sync_copy(
                    dst_hbm.at[pl.ds(s * _PROP_CH + ch0, _HALF)], dstv2)

                def duo(t, carry):
                    k0 = t * _NBUF
                    descs = [
                        pltpu.async_copy(z_hbm.at[srcv2.at[k0 + b]],
                                         rows2.at[b], sems[b])
                        for b in range(_NBUF)
                    ]
                    for b in range(_NBUF):
                        descs[b].wait()
                        pltpu.sync_copy(rows2.at[b],
                                        acc_sh.at[dstv2.at[k0 + b]],
                                        add=True)
                    return carry

                lax.fori_loop(0, _HALF // _NBUF, duo, 0)

            plsc.subcore_barrier()
            pltpu.sync_copy(acc_sh.at[pl.ds(r0, _RL)],
                            out_hbm.at[pl.ds(q * _N + r0, _RL)])
            plsc.subcore_barrier()

    return prop


def _dinv_block(deg_ref):
    d = deg_ref[0, :, 0] + deg_ref[1, :, 0] + 1.0
    return lax.rsqrt(d)


def _pre1_body(x_ref, deg_ref, we_ref, be_ref, wc_ref, out_ref):
    h0 = jnp.dot(x_ref[...], we_ref[...],
                 preferred_element_type=jnp.float32) + be_ref[...]
    z = jnp.dot(h0, wc_ref[...], preferred_element_type=jnp.float32)
    z = z * _dinv_block(deg_ref)[:, None]
    for q in range(4):
        out_ref[q] = z[:, q * 128:(q + 1) * 128]


def _pre1(x, deg3, w_enc, b_enc, wc1):
    return pl.pallas_call(
        _pre1_body,
        grid=(_G,),
        in_specs=[
            pl.BlockSpec((_R, _IN), lambda i: (i, 0)),
            pl.BlockSpec((2, _R, 128), lambda i: (0, i, 0)),
            pl.BlockSpec((_IN, _H), lambda i: (0, 0)),
            pl.BlockSpec((1, _H), lambda i: (0, 0)),
            pl.BlockSpec((_H, _H), lambda i: (0, 0)),
        ],
        out_specs=pl.BlockSpec((4, _R, 128), lambda i: (0, i, 0)),
        out_shape=jax.ShapeDtypeStruct((4, _N, 128), jnp.float32),
    )(x, deg3, w_enc, b_enc, wc1)


def _make_post(n_slices):
    hs = n_slices * 128

    def body(w_ref, deg_ref, b_ref, u_ref, stats_ref, sum_ref, sq_ref):
        i = pl.program_id(0)
        u = jnp.concatenate([w_ref[q] for q in range(n_slices)], axis=1)
        u = u * _dinv_block(deg_ref)[:, None] + b_ref[...]
        u_ref[...] = u

        @pl.when(i == 0)
        def _():
            sum_ref[...] = jnp.zeros_like(sum_ref)
            sq_ref[...] = jnp.zeros_like(sq_ref)

        sum_ref[...] = sum_ref[...] + jnp.sum(u, axis=0, keepdims=True)
        sq_ref[...] = sq_ref[...] + jnp.sum(u * u, axis=0, keepdims=True)

        @pl.when(i == _G - 1)
        def _():
            stats_ref[0:1, :] = sum_ref[...]
            stats_ref[1:2, :] = sq_ref[...]

    def post(w3d, deg3, b):
        return pl.pallas_call(
            body,
            grid=(_G,),
            in_specs=[
                pl.BlockSpec((n_slices, _R, 128), lambda i: (0, i, 0)),
                pl.BlockSpec((2, _R, 128), lambda i: (0, i, 0)),
                pl.BlockSpec((1, hs), lambda i: (0, 0)),
            ],
            out_specs=[
                pl.BlockSpec((_R, hs), lambda i: (i, 0)),
                pl.BlockSpec((2, hs), lambda i: (0, 0)),
            ],
            out_shape=[
                jax.ShapeDtypeStruct((_N, hs), jnp.float32),
                jax.ShapeDtypeStruct((2, hs), jnp.float32),
            ],
            scratch_shapes=[
                pltpu.VMEM((1, hs), jnp.float32),
                pltpu.VMEM((1, hs), jnp.float32),
            ],
        )(w3d, deg3, b)

    return post


_post4 = _make_post(4)
_post2 = _make_post(2)


def _make_pre(h_in, n_slices):
    def body(u_ref, st_ref, g_ref, bt_ref, deg_ref, w_ref, out_ref):
        st = st_ref[...]
        m = st[0:1, :] * (1.0 / _N)
        var = st[1:2, :] * (1.0 / _N) - m * m
        sc = g_ref[...] * lax.rsqrt(var + 1e-5)
        h = jnp.maximum((u_ref[...] - m) * sc + bt_ref[...], 0.0)
        z = jnp.dot(h, w_ref[...], preferred_element_type=jnp.float32)
        z = z * _dinv_block(deg_ref)[:, None]
        for q in range(n_slices):
            out_ref[q] = z[:, q * 128:(q + 1) * 128]

    def pre(u, st, g, bt, deg3, w):
        return pl.pallas_call(
            body,
            grid=(_G,),
            in_specs=[
                pl.BlockSpec((_R, h_in), lambda i: (i, 0)),
                pl.BlockSpec((2, h_in), lambda i: (0, 0)),
                pl.BlockSpec((1, h_in), lambda i: (0, 0)),
                pl.BlockSpec((1, h_in), lambda i: (0, 0)),
                pl.BlockSpec((2, _R, 128), lambda i: (0, i, 0)),
                pl.BlockSpec((h_in, n_slices * 128), lambda i: (0, 0)),
            ],
            out_specs=pl.BlockSpec((n_slices, _R, 128), lambda i: (0, i, 0)),
            out_shape=jax.ShapeDtypeStruct((n_slices, _N, 128), jnp.float32),
        )(u, st, g, bt, deg3, w)

    return pre


_pre2 = _make_pre(_H, 4)
_pre3 = _make_pre(_H, 2)


def _final_body(u_ref, st_ref, g_ref, bt_ref,
                we1_ref, be1_ref, we2t_ref, be2_ref,
                wm1_ref, bm1_ref, wm2t_ref, bm2_ref,
                wf1_ref, bf1_ref, wf2_ref, bf2_ref,
                emb_ref, ge_ref, e_ref, m_ref, f_ref, csum_ref):
    i = pl.program_id(0)
    st = st_ref[...]
    mean = st[0:1, :] * (1.0 / _N)
    var = st[1:2, :] * (1.0 / _N) - mean * mean
    sc = g_ref[...] * lax.rsqrt(var + 1e-5)
    h3 = (u_ref[...] - mean) * sc + bt_ref[...]
    emb_ref[...] = h3

    @pl.when(i == 0)
    def _():
        csum_ref[...] = jnp.zeros_like(csum_ref)

    csum_ref[...] = csum_ref[...] + jnp.sum(h3, axis=0, keepdims=True)

    @pl.when(i == _G - 1)
    def _():
        ge = csum_ref[...] * (1.0 / _N)
        ge_ref[...] = ge
        ae = jnp.maximum(jnp.dot(ge, we1_ref[...],
                                 preferred_element_type=jnp.float32)
                         + be1_ref[...], 0.0)
        e_ref[...] = jax.nn.sigmoid(
            jnp.sum(ae * we2t_ref[...], axis=1, keepdims=True) + be2_ref[...])
        am = jnp.maximum(jnp.dot(ge, wm1_ref[...],
                                 preferred_element_type=jnp.float32)
                         + bm1_ref[...], 0.0)
        m_ref[...] = jax.nn.sigmoid(
            jnp.sum(am * wm2t_ref[...], axis=1, keepdims=True) + bm2_ref[...])
        af = jnp.maximum(jnp.dot(ge, wf1_ref[...],
                                 preferred_element_type=jnp.float32)
                         + bf1_ref[...], 0.0)
        f_ref[...] = jnp.dot(af, wf2_ref[...],
                             preferred_element_type=jnp.float32) + bf2_ref[...]


def _final(u3, st3, g3, bt3, we1, be1, we2t, be2, wm1, bm1, wm2t, bm2,
           wf1, bf1, wf2, bf2):
    hh = _OUT // 2

    def full(shape):
        return pl.BlockSpec(shape, lambda i: tuple(0 for _ in shape))

    return pl.pallas_call(
        _final_body,
        grid=(_G,),
        in_specs=[
            pl.BlockSpec((_R, _OUT), lambda i: (i, 0)),
            full((2, _OUT)), full((1, _OUT)), full((1, _OUT)),
            full((_OUT, hh)), full((1, hh)), full((1, hh)), full((1, 1)),
            full((_OUT, hh)), full((1, hh)), full((1, hh)), full((1, 1)),
            full((_OUT, hh)), full((1, hh)), full((hh, 6)), full((1, 6)),
        ],
        out_specs=[
            pl.BlockSpec((_R, _OUT), lambda i: (i, 0)),
            full((1, _OUT)), full((1, 1)), full((1, 1)), full((1, 6)),
        ],
        out_shape=[
            jax.ShapeDtypeStruct((_N, _OUT), jnp.float32),
            jax.ShapeDtypeStruct((1, _OUT), jnp.float32),
            jax.ShapeDtypeStruct((1, 1), jnp.float32),
            jax.ShapeDtypeStruct((1, 1), jnp.float32),
            jax.ShapeDtypeStruct((1, 6), jnp.float32),
        ],
        scratch_shapes=[pltpu.VMEM((1, _OUT), jnp.float32)],
    )(u3, st3, g3, bt3, we1, be1, we2t, be2, wm1, bm1, wm2t, bm2,
      wf1, bf1, wf2, bf2)


def kernel(x, edge_index, W_enc, b_enc, Wc1, bc1, g1, beta1, Wc2, bc2, g2,
           beta2, Wc3, bc3, g3, beta3, We1, be1, We2, be2, Wm1, bm1, Wm2,
           bm2, Wf1, bf1, Wf2, bf2):
    pad = _EPAD - _E
    src_p = jnp.concatenate([edge_index[0], jnp.zeros((pad,), jnp.int32)])
    dst_p = jnp.concatenate([edge_index[1], jnp.full((pad,), _N, jnp.int32)])
    dst2 = dst_p.reshape(_NS * _PROP_CH, _PCHUNK)
    off = jnp.arange(4, dtype=jnp.int32)[:, None] * _N
    srcadj4 = (src_p[None, :] + off).reshape(4 * _NS * _PROP_CH, _PCHUNK)
    srcadj2 = (src_p[None, :] + off[:2]).reshape(2 * _NS * _PROP_CH, _PCHUNK)
    zeros0 = jnp.zeros((_RL, 128), jnp.float32)
    ones0 = jnp.ones((_CHUNK, 128), jnp.float32)

    def row(v):
        return v.reshape(1, -1)

    deg2 = _deg()(dst_p, zeros0, ones0)
    deg3 = deg2.reshape(2, _N, 128)

    z1 = _pre1(x, deg3, W_enc, row(b_enc), Wc1)
    w1 = _prop(4)(srcadj4, dst2, z1.reshape(4 * _N, 128))
    u1, st1 = _post4(w1.reshape(4, _N, 128), deg3, row(bc1))

    z2 = _pre2(u1, st1, row(g1), row(beta1), deg3, Wc2)
    w2 = _prop(4)(srcadj4, dst2, z2.reshape(4 * _N, 128))
    u2, st2 = _post4(w2.reshape(4, _N, 128), deg3, row(bc2))

    z3 = _pre3(u2, st2, row(g2), row(beta2), deg3, Wc3)
    w3 = _prop(2)(srcadj2, dst2, z3.reshape(2 * _N, 128))
    u3, st3 = _post2(w3.reshape(2, _N, 128), deg3, row(bc3))

    emb, ge, e, m, f = _final(
        u3, st3, row(g3), row(beta3),
        We1, row(be1), We2.reshape(1, -1), row(be2),
        Wm1, row(bm1), Wm2.reshape(1, -1), row(bm2),
        Wf1, row(bf1), Wf2, row(bf2))
    return (emb, ge, e, m, f)
